# Initial kernel scaffold; baseline (speedup 1.0000x reference)
import jax
import jax.numpy as jnp
from jax.experimental import pallas as pl


def kernel(x):
    raise NotImplementedError("write your pallas kernel here")


# speedup vs baseline: 7.4448x; 7.4448x over previous
import functools

import jax
import jax.numpy as jnp
from jax import lax
from jax.experimental import pallas as pl
from jax.experimental.pallas import tpu as pltpu
from jax.experimental.pallas import tpu_sc as plsc

B, C, N, K = 4, 128, 2000, 9
NQ = 2048
BQ = 256

_NC, _NS = 2, 16
_NW = _NC * _NS
_TPW = (B * C) // _NW
_LANES = 16


def _topk_body(x_ref, q_ref, idx_ref):
    X = x_ref[0]
    Q = q_ref[0]
    xx = jnp.sum(X * X, axis=0)
    qq = jnp.sum(Q * Q, axis=0)
    qX = lax.dot_general(
        Q, X, (((0,), (0,)), ((), ())), preferred_element_type=jnp.float32
    )
    s = 2.0 * qX - qq[:, None] - xx[None, :]
    kio = lax.broadcasted_iota(jnp.int32, (BQ, N), 1)
    big = jnp.int32(2**30)
    for j in range(K):
        m = jnp.max(s, axis=1, keepdims=True)
        cand = jnp.where(s == m, kio, big)
        a = jnp.min(cand, axis=1)
        idx_ref[0, j, :] = a
        s = jnp.where(kio == a[:, None], -jnp.inf, s)


def _topk_call(x):
    xq = jnp.pad(x, ((0, 0), (0, 0), (0, NQ - N)))
    idx = pl.pallas_call(
        _topk_body,
        grid=(B, NQ // BQ),
        in_specs=[
            pl.BlockSpec((1, C, N), lambda b, q: (b, 0, 0)),
            pl.BlockSpec((1, C, BQ), lambda b, q: (b, 0, q)),
        ],
        out_specs=pl.BlockSpec((1, K, BQ), lambda b, q: (b, 0, q)),
        out_shape=jax.ShapeDtypeStruct((B, K, NQ), jnp.int32),
    )(x, xq)
    return idx[:, :, :N]


def _gather_body(x_hbm, idx_hbm, rep_hbm, out_hbm, idx_v, rep_v, xrow_v, o1_v, o2_v):
    cid = lax.axis_index("c")
    sid = lax.axis_index("s")
    wid = sid * _NC + cid
    t0 = wid * _TPW
    b = t0 // C
    c0 = t0 % C
    pltpu.sync_copy(rep_hbm, rep_v)
    pltpu.sync_copy(idx_hbm.at[b], idx_v)

    def task(ci, carry):
        c = c0 + ci
        pltpu.sync_copy(x_hbm.at[b, c], xrow_v)

        def inner(t, carry2):
            sl = pl.ds(pl.multiple_of(t * _LANES, 8), _LANES)
            ii = idx_v[sl]
            rr = rep_v[sl]
            xn = plsc.load_gather(xrow_v, [ii])
            xr = plsc.load_gather(xrow_v, [rr])
            o1_v[sl] = xr
            o2_v[sl] = xr - xn
            return carry2

        lax.fori_loop(0, (N * K) // _LANES, inner, 0)
        pltpu.sync_copy(o1_v, out_hbm.at[b, c])
        pltpu.sync_copy(o2_v, out_hbm.at[b, c + C])
        return carry

    lax.fori_loop(0, _TPW, task, 0)


def _gather_call(x, idx_t, rep):
    mesh = plsc.VectorSubcoreMesh(
        core_axis_name="c", subcore_axis_name="s", num_cores=_NC, num_subcores=_NS
    )
    f = pl.kernel(
        _gather_body,
        out_type=jax.ShapeDtypeStruct((B, 2 * C, N * K), jnp.float32),
        mesh=mesh,
        scratch_types=[
            pltpu.VMEM((N * K,), jnp.int32),
            pltpu.VMEM((N * K,), jnp.int32),
            pltpu.VMEM((N,), jnp.float32),
            pltpu.VMEM((N * K,), jnp.float32),
            pltpu.VMEM((N * K,), jnp.float32),
        ],
        compiler_params=pltpu.CompilerParams(needs_layout_passes=False),
    )
    return f(x, idx_t, rep)


def kernel(x):
    idx = _topk_call(x)
    idx_t = jnp.transpose(idx, (0, 2, 1)).reshape(B, N * K)
    rep = jnp.repeat(jnp.arange(N, dtype=jnp.int32), K)
    out = _gather_call(x, idx_t, rep)
    return out.reshape(B, 2 * C, N, K)


# speedup vs baseline: 7.4859x; 1.0055x over previous
import functools

import jax
import jax.numpy as jnp
from jax import lax
from jax.experimental import pallas as pl
from jax.experimental.pallas import tpu as pltpu
from jax.experimental.pallas import tpu_sc as plsc

B, C, N, K = 4, 128, 2000, 9
NQ = 2048
BQ = 256

_NC, _NS = 2, 16
_NW = _NC * _NS
_TPW = (B * C) // _NW
_LANES = 16


def _topk_body(x_ref, q_ref, idx_ref):
    X = x_ref[0]
    Q = q_ref[0]
    xx = jnp.sum(X * X, axis=0)
    qq = jnp.sum(Q * Q, axis=0)
    qX = lax.dot_general(
        Q, X, (((0,), (0,)), ((), ())), preferred_element_type=jnp.float32
    )
    s = 2.0 * qX - qq[:, None] - xx[None, :]
    kio = lax.broadcasted_iota(jnp.int32, (BQ, N), 1)
    big = jnp.int32(2**30)
    for j in range(K):
        m = jnp.max(s, axis=1, keepdims=True)
        cand = jnp.where(s == m, kio, big)
        a = jnp.min(cand, axis=1)
        idx_ref[0, j, :] = a
        s = jnp.where(kio == a[:, None], -jnp.inf, s)


def _topk_call(x):
    return pl.pallas_call(
        _topk_body,
        grid=(B, NQ // BQ),
        in_specs=[
            pl.BlockSpec((1, C, N), lambda b, q: (b, 0, 0)),
            pl.BlockSpec((1, C, BQ), lambda b, q: (b, 0, q)),
        ],
        out_specs=pl.BlockSpec((1, K, BQ), lambda b, q: (b, 0, q)),
        out_shape=jax.ShapeDtypeStruct((B, K, N), jnp.int32),
    )(x, x)


def _gather_body(x_hbm, idx_hbm, rep_hbm, out_hbm, idx_v, rep_v, xrow_v, o1_v, o2_v):
    cid = lax.axis_index("c")
    sid = lax.axis_index("s")
    wid = sid * _NC + cid
    t0 = wid * _TPW
    b = t0 // C
    c0 = t0 % C
    pltpu.sync_copy(rep_hbm, rep_v)
    pltpu.sync_copy(idx_hbm.at[b], idx_v)

    def task(ci, carry):
        c = c0 + ci
        pltpu.sync_copy(x_hbm.at[b, c], xrow_v)

        def inner(t, carry2):
            sl = pl.ds(pl.multiple_of(t * _LANES, 8), _LANES)
            ii = idx_v[sl]
            rr = rep_v[sl]
            xn = plsc.load_gather(xrow_v, [ii])
            xr = plsc.load_gather(xrow_v, [rr])
            o1_v[sl] = xr
            o2_v[sl] = xr - xn
            return carry2

        lax.fori_loop(0, (N * K) // _LANES, inner, 0)
        pltpu.sync_copy(o1_v, out_hbm.at[b, c])
        pltpu.sync_copy(o2_v, out_hbm.at[b, c + C])
        return carry

    lax.fori_loop(0, _TPW, task, 0)


def _gather_call(x, idx_t, rep):
    mesh = plsc.VectorSubcoreMesh(
        core_axis_name="c", subcore_axis_name="s", num_cores=_NC, num_subcores=_NS
    )
    f = pl.kernel(
        _gather_body,
        out_type=jax.ShapeDtypeStruct((B, 2 * C, N * K), jnp.float32),
        mesh=mesh,
        scratch_types=[
            pltpu.VMEM((N * K,), jnp.int32),
            pltpu.VMEM((N * K,), jnp.int32),
            pltpu.VMEM((N,), jnp.float32),
            pltpu.VMEM((N * K,), jnp.float32),
            pltpu.VMEM((N * K,), jnp.float32),
        ],
        compiler_params=pltpu.CompilerParams(needs_layout_passes=False),
    )
    return f(x, idx_t, rep)


def kernel(x):
    idx = _topk_call(x)
    idx_t = jnp.transpose(idx, (0, 2, 1)).reshape(B, N * K)
    rep = jnp.repeat(jnp.arange(N, dtype=jnp.int32), K)
    out = _gather_call(x, idx_t, rep)
    return out.reshape(B, 2 * C, N, K)


# speedup vs baseline: 9.2444x; 1.2349x over previous
import functools

import jax
import jax.numpy as jnp
from jax import lax
from jax.experimental import pallas as pl
from jax.experimental.pallas import tpu as pltpu
from jax.experimental.pallas import tpu_sc as plsc

B, C, N, K = 4, 128, 2000, 9
NQ = 2048
BQ = 256

_NC, _NS = 2, 16
_NW = _NC * _NS
_TPW = (B * C) // _NW
_LANES = 16


def _topk_body(x_ref, q_ref, idx_ref):
    X = x_ref[0]
    Q = q_ref[0]
    xx = jnp.sum(X * X, axis=0)
    qq = jnp.sum(Q * Q, axis=0)
    qX = lax.dot_general(
        Q, X, (((0,), (0,)), ((), ())), preferred_element_type=jnp.float32
    )
    s = 2.0 * qX - qq[:, None] - xx[None, :]
    kio = lax.broadcasted_iota(jnp.int32, (BQ, N), 1)
    big = jnp.int32(2**30)
    for j in range(K):
        m = jnp.max(s, axis=1, keepdims=True)
        cand = jnp.where(s == m, kio, big)
        a = jnp.min(cand, axis=1)
        idx_ref[0, j, :] = a
        s = jnp.where(kio == a[:, None], -jnp.inf, s)


def _topk_call(x):
    return pl.pallas_call(
        _topk_body,
        grid=(B, NQ // BQ),
        in_specs=[
            pl.BlockSpec((1, C, N), lambda b, q: (b, 0, 0)),
            pl.BlockSpec((1, C, BQ), lambda b, q: (b, 0, q)),
        ],
        out_specs=pl.BlockSpec((1, K, BQ), lambda b, q: (b, 0, q)),
        out_shape=jax.ShapeDtypeStruct((B, K, N), jnp.int32),
    )(x, x)


def _task_compute(x_v, idxT_v, o1_v, o2_v):
    vi9 = lax.broadcasted_iota(jnp.int32, (_LANES,), 0) * 9

    def body_t(t):
        base = pl.multiple_of(t * _LANES, 8)
        xr = x_v[pl.ds(base, _LANES)]
        pos0 = vi9 + t * (_LANES * K)
        for j in range(K):
            nn = idxT_v[j, pl.ds(base, _LANES)]
            xn = plsc.load_gather(x_v, [nn])
            pos = pos0 + j
            plsc.store_scatter(o1_v, [pos], xr)
            plsc.store_scatter(o2_v, [pos], xr - xn)

    plsc.parallel_loop(0, N // _LANES, 1, unroll=2, carry=None)(body_t)


def _gather_body(x_hbm, idx_hbm, out_hbm, idxT_v, x_v, o1a, o2a, o1b, o2b, s0, s1):
    cid = lax.axis_index("c")
    sid = lax.axis_index("s")
    wid = sid * _NC + cid
    t0 = wid * _TPW
    b = t0 // C
    c0 = t0 % C
    pltpu.sync_copy(idx_hbm.at[b], idxT_v)
    obufs = ((o1a, o2a, s0), (o1b, o2b, s1))
    pending = [None, None]
    for ci in range(_TPW):
        o1_v, o2_v, sem = obufs[ci % 2]
        c = c0 + ci
        pltpu.sync_copy(x_hbm.at[b, c], x_v)
        if pending[ci % 2] is not None:
            for h in pending[ci % 2]:
                h.wait()
        _task_compute(x_v, idxT_v, o1_v, o2_v)
        h1 = pltpu.async_copy(o1_v, out_hbm.at[b, c], sem)
        h2 = pltpu.async_copy(o2_v, out_hbm.at[b, c + C], sem)
        pending[ci % 2] = (h1, h2)
    for pair in pending:
        for h in pair:
            h.wait()


def _gather_call(x, idx):
    mesh = plsc.VectorSubcoreMesh(
        core_axis_name="c", subcore_axis_name="s", num_cores=_NC, num_subcores=_NS
    )
    f = pl.kernel(
        _gather_body,
        out_type=jax.ShapeDtypeStruct((B, 2 * C, N * K), jnp.float32),
        mesh=mesh,
        scratch_types=[
            pltpu.VMEM((K, N), jnp.int32),
            pltpu.VMEM((N,), jnp.float32),
            pltpu.VMEM((N * K,), jnp.float32),
            pltpu.VMEM((N * K,), jnp.float32),
            pltpu.VMEM((N * K,), jnp.float32),
            pltpu.VMEM((N * K,), jnp.float32),
            pltpu.SemaphoreType.DMA,
            pltpu.SemaphoreType.DMA,
        ],
        compiler_params=pltpu.CompilerParams(needs_layout_passes=False),
    )
    return f(x, idx)


def kernel(x):
    idx = _topk_call(x)
    out = _gather_call(x, idx)
    return out.reshape(B, 2 * C, N, K)


# speedup vs baseline: 12.2071x; 1.3205x over previous
import functools

import jax
import jax.numpy as jnp
from jax import lax
from jax.experimental import pallas as pl
from jax.experimental.pallas import tpu as pltpu
from jax.experimental.pallas import tpu_sc as plsc

B, C, N, K = 4, 128, 2000, 9
NQ = 2048
BQ = 256

_NC, _NS = 2, 16
_NW = _NC * _NS
_LANES = 16
_CH = 80
_NCHUNK = N // _CH
_NU = B * K * _NCHUNK


def _topk_body(x_ref, q_ref, idx_ref):
    X = x_ref[0]
    Q = q_ref[0]
    xx = jnp.sum(X * X, axis=0)
    qq = jnp.sum(Q * Q, axis=0)
    qX = lax.dot_general(
        Q, X, (((0,), (0,)), ((), ())), preferred_element_type=jnp.float32
    )
    s = 2.0 * qX - qq[:, None] - xx[None, :]
    kio = lax.broadcasted_iota(jnp.int32, (BQ, N), 1).astype(jnp.float32)
    for j in range(K):
        m = jnp.max(s, axis=1, keepdims=True)
        cand = jnp.where(s == m, kio, jnp.float32(N))
        a = jnp.min(cand, axis=1)
        idx_ref[0, :, j] = a.astype(jnp.int32)
        s = jnp.where(kio == a[:, None], -jnp.inf, s)


def _topk_call(x):
    return pl.pallas_call(
        _topk_body,
        grid=(B, NQ // BQ),
        in_specs=[
            pl.BlockSpec((1, C, N), lambda b, q: (b, 0, 0)),
            pl.BlockSpec((1, C, BQ), lambda b, q: (b, 0, q)),
        ],
        out_specs=pl.BlockSpec((1, BQ, 16), lambda b, q: (b, q, 0)),
        out_shape=jax.ShapeDtypeStruct((B, N, 16), jnp.int32),
    )(x, x)


def _tr_body(x_ref, o_ref):
    o_ref[0] = x_ref[0].T


def _tr_call(x):
    return pl.pallas_call(
        _tr_body,
        grid=(B,),
        in_specs=[pl.BlockSpec((1, C, N), lambda b: (b, 0, 0))],
        out_specs=pl.BlockSpec((1, N, C), lambda b: (b, 0, 0)),
        out_shape=jax.ShapeDtypeStruct((B, N, C), jnp.float32),
    )(x)


def _unit(p, u, xT_hbm, idx_hbm, out_hbm, idxc, idxl, xrows, grows, ob, sg, so,
          first):
    b = u // (K * _NCHUNK)
    r0 = u % (K * _NCHUNK)
    j = r0 // _NCHUNK
    n0 = pl.multiple_of((r0 % _NCHUNK) * _CH, 8)
    pltpu.sync_copy(idx_hbm.at[b, pl.ds(n0, _CH)], idxc[p])
    pltpu.sync_copy(xT_hbm.at[b, pl.ds(n0, _CH)], xrows[p])
    rio = lax.broadcasted_iota(jnp.int32, (_LANES,), 0)
    vj = jnp.full((_LANES,), 0, jnp.int32) + j
    for h in range(_CH // _LANES):
        rows = rio + (h * _LANES)
        col = plsc.load_gather(idxc[p], [rows, vj])
        idxl[p][pl.ds(h * _LANES, _LANES)] = col
    pltpu.async_copy(xT_hbm.at[b].at[idxl[p]], grows[p], sg[p]).wait()

    @pl.when(jnp.logical_not(first))
    def _drain():
        pltpu.make_async_copy(
            ob[p], out_hbm.at[0, 0, pl.ds(0, _CH)], so[p]
        ).wait()

    def row(r):
        for h in range(C // _LANES):
            l0 = h * _LANES
            xv = xrows[p][r, pl.ds(l0, _LANES)]
            gv = grows[p][r, pl.ds(l0, _LANES)]
            ob[p][r, pl.ds(l0, _LANES)] = xv
            ob[p][r, pl.ds(C + l0, _LANES)] = xv - gv

    plsc.parallel_loop(0, _CH, 1, unroll=2)(row)
    pltpu.async_copy(ob[p], out_hbm.at[b, j, pl.ds(n0, _CH)], so[p])


def _gather_body(xT_hbm, idx_hbm, out_hbm, idxc0, idxc1, idxl0, idxl1, xr0, xr1,
                 gr0, gr1, ob0, ob1, sg0, sg1, so0, so1):
    cid = lax.axis_index("c")
    sid = lax.axis_index("s")
    w = sid * _NC + cid
    idxc = (idxc0, idxc1)
    idxl = (idxl0, idxl1)
    xrows = (xr0, xr1)
    grows = (gr0, gr1)
    ob = (ob0, ob1)
    sg = (sg0, sg1)
    so = (so0, so1)

    def step(i, carry):
        for p in range(2):
            ui = i * 2 + p
            u = w + ui * _NW

            @pl.when(u < _NU)
            def _do():
                _unit(p, u, xT_hbm, idx_hbm, out_hbm, idxc, idxl, xrows, grows,
                      ob, sg, so, first=(ui == p))

        return carry

    lax.fori_loop(0, (_NU // _NW + 2) // 2 + 1, step, 0)
    for p in range(2):
        pltpu.make_async_copy(ob[p], out_hbm.at[0, 0, pl.ds(0, _CH)], so[p]).wait()


def _gather_call(xT, idx):
    mesh = plsc.VectorSubcoreMesh(
        core_axis_name="c", subcore_axis_name="s", num_cores=_NC, num_subcores=_NS
    )
    f = pl.kernel(
        _gather_body,
        out_type=jax.ShapeDtypeStruct((B, K, N, 2 * C), jnp.float32),
        mesh=mesh,
        scratch_types=[
            pltpu.VMEM((_CH, 16), jnp.int32),
            pltpu.VMEM((_CH, 16), jnp.int32),
            pltpu.VMEM((_CH,), jnp.int32),
            pltpu.VMEM((_CH,), jnp.int32),
            pltpu.VMEM((_CH, C), jnp.float32),
            pltpu.VMEM((_CH, C), jnp.float32),
            pltpu.VMEM((_CH, C), jnp.float32),
            pltpu.VMEM((_CH, C), jnp.float32),
            pltpu.VMEM((_CH, 2 * C), jnp.float32),
            pltpu.VMEM((_CH, 2 * C), jnp.float32),
            pltpu.SemaphoreType.DMA,
            pltpu.SemaphoreType.DMA,
            pltpu.SemaphoreType.DMA,
            pltpu.SemaphoreType.DMA,
        ],
        compiler_params=pltpu.CompilerParams(needs_layout_passes=False),
    )
    return f(xT, idx)


def kernel(x):
    idx = _topk_call(x)
    xT = _tr_call(x)
    out = _gather_call(xT, idx)
    return jnp.transpose(out, (0, 3, 2, 1))


# speedup vs baseline: 13.6212x; 1.1158x over previous
import functools

import jax
import jax.numpy as jnp
from jax import lax
from jax.experimental import pallas as pl
from jax.experimental.pallas import tpu as pltpu
from jax.experimental.pallas import tpu_sc as plsc

B, C, N, K = 4, 128, 2000, 9
NQ = 2048
BQ = 256

_NC, _NS = 2, 16
_NW = _NC * _NS
_LANES = 16
_CH = 16
_NCHUNK = N // _CH
_NU = B * _NCHUNK


def _topk_body(x_ref, q_ref, idx_ref):
    X = x_ref[0]
    Q = q_ref[0]
    xx = jnp.sum(X * X, axis=0)
    qq = jnp.sum(Q * Q, axis=0)
    qX = lax.dot_general(
        Q, X, (((0,), (0,)), ((), ())), preferred_element_type=jnp.float32
    )
    s = 2.0 * qX - qq[:, None] - xx[None, :]
    kio = lax.broadcasted_iota(jnp.int32, (BQ, N), 1).astype(jnp.float32)
    for j in range(K):
        m = jnp.max(s, axis=1, keepdims=True)
        cand = jnp.where(s == m, kio, jnp.float32(N))
        a = jnp.min(cand, axis=1)
        idx_ref[0, :, j] = a.astype(jnp.int32)
        s = jnp.where(kio == a[:, None], -jnp.inf, s)


def _topk_call(x):
    return pl.pallas_call(
        _topk_body,
        grid=(B, NQ // BQ),
        in_specs=[
            pl.BlockSpec((1, C, N), lambda b, q: (b, 0, 0)),
            pl.BlockSpec((1, C, BQ), lambda b, q: (b, 0, q)),
        ],
        out_specs=pl.BlockSpec((1, BQ, 16), lambda b, q: (b, q, 0)),
        out_shape=jax.ShapeDtypeStruct((B, N, 16), jnp.int32),
    )(x, x)


def _tr_body(x_ref, o_ref):
    o_ref[0] = x_ref[0].T


def _tr_call(x):
    return pl.pallas_call(
        _tr_body,
        grid=(B,),
        in_specs=[pl.BlockSpec((1, C, N), lambda b: (b, 0, 0))],
        out_specs=pl.BlockSpec((1, N, C), lambda b: (b, 0, 0)),
        out_shape=jax.ShapeDtypeStruct((B, N, C), jnp.float32),
    )(x)


def _prefetch(p, u, xT_hbm, idx_hbm, idxc, xrows, sin):
    b = u // _NCHUNK
    n0 = pl.multiple_of((u % _NCHUNK) * _CH, 8)
    pltpu.async_copy(idx_hbm.at[b, pl.ds(n0, _CH)], idxc[p], sin[p])
    pltpu.async_copy(xT_hbm.at[b, pl.ds(n0, _CH)], xrows[p], sin[p])


def _unit(p, ui, u, xT_hbm, idx_hbm, out_hbm, idxc, idxl, xrows, grows, obr,
          sin, sg, so):
    b = u // _NCHUNK
    n0 = pl.multiple_of((u % _NCHUNK) * _CH, 8)
    pltpu.make_async_copy(idx_hbm.at[0, pl.ds(0, _CH)], idxc[p], sin[p]).wait()
    pltpu.make_async_copy(xT_hbm.at[0, pl.ds(0, _CH)], xrows[p], sin[p]).wait()
    nxt = u + _NW

    @pl.when(nxt < _NU)
    def _pf():
        _prefetch(1 - p, nxt, xT_hbm, idx_hbm, idxc, xrows, sin)

    rio = lax.broadcasted_iota(jnp.int32, (_LANES,), 0)
    for j in range(K):
        vj = jnp.full((_LANES,), j, jnp.int32)
        idxl[p][j, pl.ds(0, _LANES)] = plsc.load_gather(idxc[p], [rio, vj])
    for j in range(K):
        pltpu.async_copy(xT_hbm.at[b].at[idxl[p].at[j]], grows[p].at[j], sg[p])
    for j in range(K):
        pltpu.make_async_copy(
            xT_hbm.at[b, pl.ds(0, _CH)], grows[p].at[j], sg[p]
        ).wait()
    for j in range(K):
        sl = j % 4
        if j >= 4:
            pltpu.make_async_copy(
                obr[sl], out_hbm.at[0, 0, pl.ds(0, _CH)], so[sl]
            ).wait()
        else:

            @pl.when(ui > 0)
            def _drain():
                pltpu.make_async_copy(
                    obr[sl], out_hbm.at[0, 0, pl.ds(0, _CH)], so[sl]
                ).wait()

        def row(r, j=j, sl=sl):
            for h in range(C // _LANES):
                l0 = h * _LANES
                xv = xrows[p][r, pl.ds(l0, _LANES)]
                gv = grows[p][j, r, pl.ds(l0, _LANES)]
                obr[sl][r, pl.ds(l0, _LANES)] = xv
                obr[sl][r, pl.ds(C + l0, _LANES)] = xv - gv

        plsc.parallel_loop(0, _CH, 1, unroll=2)(row)
        pltpu.async_copy(obr[sl], out_hbm.at[b, j, pl.ds(n0, _CH)], so[sl])


def _gather_body(xT_hbm, idx_hbm, out_hbm, idxc0, idxc1, idxl0, idxl1, xr0, xr1,
                 gr0, gr1, ob0, ob1, ob2, ob3, sin0, sin1, sg0, sg1,
                 so0, so1, so2, so3):
    cid = lax.axis_index("c")
    sid = lax.axis_index("s")
    w = sid * _NC + cid
    idxc = (idxc0, idxc1)
    idxl = (idxl0, idxl1)
    xrows = (xr0, xr1)
    grows = (gr0, gr1)
    obr = (ob0, ob1, ob2, ob3)
    sin = (sin0, sin1)
    sg = (sg0, sg1)
    so = (so0, so1, so2, so3)

    _prefetch(0, w, xT_hbm, idx_hbm, idxc, xrows, sin)

    def step(i, carry):
        for p in range(2):
            ui = i * 2 + p
            u = w + ui * _NW

            @pl.when(u < _NU)
            def _do():
                _unit(p, ui, u, xT_hbm, idx_hbm, out_hbm, idxc, idxl, xrows,
                      grows, obr, sin, sg, so)

        return carry

    lax.fori_loop(0, (_NU // _NW + 1 + 1) // 2, step, 0)
    for sl in range(4):
        pltpu.make_async_copy(
            obr[sl], out_hbm.at[0, 0, pl.ds(0, _CH)], so[sl]
        ).wait()


def _gather_call(xT, idx):
    mesh = plsc.VectorSubcoreMesh(
        core_axis_name="c", subcore_axis_name="s", num_cores=_NC, num_subcores=_NS
    )
    f = pl.kernel(
        _gather_body,
        out_type=jax.ShapeDtypeStruct((B, K, N, 2 * C), jnp.float32),
        mesh=mesh,
        scratch_types=[
            pltpu.VMEM((_CH, 16), jnp.int32),
            pltpu.VMEM((_CH, 16), jnp.int32),
            pltpu.VMEM((K, _LANES), jnp.int32),
            pltpu.VMEM((K, _LANES), jnp.int32),
            pltpu.VMEM((_CH, C), jnp.float32),
            pltpu.VMEM((_CH, C), jnp.float32),
            pltpu.VMEM((K, _CH, C), jnp.float32),
            pltpu.VMEM((K, _CH, C), jnp.float32),
            pltpu.VMEM((_CH, 2 * C), jnp.float32),
            pltpu.VMEM((_CH, 2 * C), jnp.float32),
            pltpu.VMEM((_CH, 2 * C), jnp.float32),
            pltpu.VMEM((_CH, 2 * C), jnp.float32),
            pltpu.SemaphoreType.DMA,
            pltpu.SemaphoreType.DMA,
            pltpu.SemaphoreType.DMA,
            pltpu.SemaphoreType.DMA,
            pltpu.SemaphoreType.DMA,
            pltpu.SemaphoreType.DMA,
            pltpu.SemaphoreType.DMA,
            pltpu.SemaphoreType.DMA,
        ],
        compiler_params=pltpu.CompilerParams(needs_layout_passes=False),
    )
    return f(xT, idx)


def kernel(x):
    idx = _topk_call(x)
    xT = _tr_call(x)
    out = _gather_call(xT, idx)
    return jnp.transpose(out, (0, 3, 2, 1))


# speedup vs baseline: 14.5192x; 1.0659x over previous
import functools

import jax
import jax.numpy as jnp
from jax import lax
from jax.experimental import pallas as pl
from jax.experimental.pallas import tpu as pltpu
from jax.experimental.pallas import tpu_sc as plsc

B, C, N, K = 4, 128, 2000, 9
NQ = 2048
BQ = 256

_NC, _NS = 2, 16
_NW = _NC * _NS
_LANES = 16
_CH = 40
_NCHUNK = N // _CH
_NU = B * _NCHUNK


def _topk_body(x_ref, q_ref, idx_ref):
    X = x_ref[0]
    Q = q_ref[0]
    xx = jnp.sum(X * X, axis=0)
    qq = jnp.sum(Q * Q, axis=0)
    qX = lax.dot_general(
        Q, X, (((0,), (0,)), ((), ())), preferred_element_type=jnp.float32
    )
    s = 2.0 * qX - qq[:, None] - xx[None, :]
    kio = lax.broadcasted_iota(jnp.int32, (BQ, N), 1).astype(jnp.float32)
    for j in range(K):
        m = jnp.max(s, axis=1, keepdims=True)
        cand = jnp.where(s == m, kio, jnp.float32(N))
        a = jnp.min(cand, axis=1)
        idx_ref[0, :, j] = a.astype(jnp.int32)
        s = jnp.where(kio == a[:, None], -jnp.inf, s)


def _topk_call(x):
    return pl.pallas_call(
        _topk_body,
        grid=(B, NQ // BQ),
        in_specs=[
            pl.BlockSpec((1, C, N), lambda b, q: (b, 0, 0)),
            pl.BlockSpec((1, C, BQ), lambda b, q: (b, 0, q)),
        ],
        out_specs=pl.BlockSpec((1, BQ, 16), lambda b, q: (b, q, 0)),
        out_shape=jax.ShapeDtypeStruct((B, N, 16), jnp.int32),
    )(x, x)


def _tr_body(x_ref, o_ref):
    o_ref[0] = x_ref[0].T


def _tr_call(x):
    return pl.pallas_call(
        _tr_body,
        grid=(B,),
        in_specs=[pl.BlockSpec((1, C, N), lambda b: (b, 0, 0))],
        out_specs=pl.BlockSpec((1, N, C), lambda b: (b, 0, 0)),
        out_shape=jax.ShapeDtypeStruct((B, N, C), jnp.float32),
    )(x)


def _prefetch(p, u, xT_hbm, idx_hbm, idxc, xrows, sin):
    b = u // _NCHUNK
    n0 = pl.multiple_of((u % _NCHUNK) * _CH, 8)
    pltpu.async_copy(idx_hbm.at[b, pl.ds(n0, _CH)], idxc[p], sin[p])
    pltpu.async_copy(xT_hbm.at[b, pl.ds(n0, _CH)], xrows[p], sin[p])


def _unit(p, ui, u, xT_hbm, idx_hbm, out_hbm, idxc, idxl, xrows, grows, obr,
          sin, sg, so):
    b = u // _NCHUNK
    n0 = pl.multiple_of((u % _NCHUNK) * _CH, 8)
    pltpu.make_async_copy(idx_hbm.at[0, pl.ds(0, _CH)], idxc[p], sin[p]).wait()
    pltpu.make_async_copy(xT_hbm.at[0, pl.ds(0, _CH)], xrows[p], sin[p]).wait()
    nxt = u + _NW

    @pl.when(nxt < _NU)
    def _pf():
        _prefetch(1 - p, nxt, xT_hbm, idx_hbm, idxc, xrows, sin)

    rio = lax.broadcasted_iota(jnp.int32, (_LANES,), 0)
    mtail = rio < jnp.full((_LANES,), _CH % _LANES, jnp.int32)
    for j in range(K):
        vj = jnp.full((_LANES,), j, jnp.int32)
        for h in range(_CH // _LANES):
            rows = rio + (h * _LANES)
            idxl[p][j, pl.ds(h * _LANES, _LANES)] = plsc.load_gather(
                idxc[p], [rows, vj]
            )
        if _CH % _LANES:
            h = _CH // _LANES
            rows = rio + (h * _LANES)
            col = plsc.load_gather(idxc[p], [rows, vj], mask=mtail)
            plsc.store_scatter(idxl[p].at[j], [rows], col, mask=mtail)
    for j in range(K):
        pltpu.async_copy(xT_hbm.at[b].at[idxl[p].at[j]], grows.at[j], sg[j])
    for j in range(K):
        pltpu.make_async_copy(
            xT_hbm.at[b, pl.ds(0, _CH)], grows.at[j], sg[j]
        ).wait()
        sl = j % 4
        if j >= 4:
            pltpu.make_async_copy(
                obr[sl], out_hbm.at[0, 0, pl.ds(0, _CH)], so[sl]
            ).wait()
        else:

            @pl.when(ui > 0)
            def _drain():
                pltpu.make_async_copy(
                    obr[sl], out_hbm.at[0, 0, pl.ds(0, _CH)], so[sl]
                ).wait()

        def row(r, j=j, sl=sl):
            for h in range(C // _LANES):
                l0 = h * _LANES
                xv = xrows[p][r, pl.ds(l0, _LANES)]
                gv = grows[j, r, pl.ds(l0, _LANES)]
                obr[sl][r, pl.ds(l0, _LANES)] = xv
                obr[sl][r, pl.ds(C + l0, _LANES)] = xv - gv

        plsc.parallel_loop(0, _CH, 1, unroll=2)(row)
        pltpu.async_copy(obr[sl], out_hbm.at[b, j, pl.ds(n0, _CH)], so[sl])


def _gather_body(xT_hbm, idx_hbm, out_hbm, idxc0, idxc1, idxl0, idxl1, xr0, xr1,
                 grows, ob0, ob1, ob2, ob3, sin0, sin1, sg0, sg1, sg2, sg3, sg4,
                 sg5, sg6, sg7, sg8, so0, so1, so2, so3):
    cid = lax.axis_index("c")
    sid = lax.axis_index("s")
    w = sid * _NC + cid
    idxc = (idxc0, idxc1)
    idxl = (idxl0, idxl1)
    xrows = (xr0, xr1)
    obr = (ob0, ob1, ob2, ob3)
    sin = (sin0, sin1)
    sg = (sg0, sg1, sg2, sg3, sg4, sg5, sg6, sg7, sg8)
    so = (so0, so1, so2, so3)

    _prefetch(0, w, xT_hbm, idx_hbm, idxc, xrows, sin)

    def step(i, carry):
        for p in range(2):
            ui = i * 2 + p
            u = w + ui * _NW

            @pl.when(u < _NU)
            def _do():
                _unit(p, ui, u, xT_hbm, idx_hbm, out_hbm, idxc, idxl, xrows,
                      grows, obr, sin, sg, so)

        return carry

    lax.fori_loop(0, (_NU // _NW + 1 + 1) // 2, step, 0)
    for sl in range(4):
        pltpu.make_async_copy(
            obr[sl], out_hbm.at[0, 0, pl.ds(0, _CH)], so[sl]
        ).wait()


def _gather_call(xT, idx):
    mesh = plsc.VectorSubcoreMesh(
        core_axis_name="c", subcore_axis_name="s", num_cores=_NC, num_subcores=_NS
    )
    f = pl.kernel(
        _gather_body,
        out_type=jax.ShapeDtypeStruct((B, K, N, 2 * C), jnp.float32),
        mesh=mesh,
        scratch_types=[
            pltpu.VMEM((_CH, 16), jnp.int32),
            pltpu.VMEM((_CH, 16), jnp.int32),
            pltpu.VMEM((K, _CH), jnp.int32),
            pltpu.VMEM((K, _CH), jnp.int32),
            pltpu.VMEM((_CH, C), jnp.float32),
            pltpu.VMEM((_CH, C), jnp.float32),
            pltpu.VMEM((K, _CH, C), jnp.float32),
            pltpu.VMEM((_CH, 2 * C), jnp.float32),
            pltpu.VMEM((_CH, 2 * C), jnp.float32),
            pltpu.VMEM((_CH, 2 * C), jnp.float32),
            pltpu.VMEM((_CH, 2 * C), jnp.float32),
        ] + [pltpu.SemaphoreType.DMA] * 15,
        compiler_params=pltpu.CompilerParams(needs_layout_passes=False),
    )
    return f(xT, idx)


def kernel(x):
    idx = _topk_call(x)
    xT = _tr_call(x)
    out = _gather_call(xT, idx)
    return jnp.transpose(out, (0, 3, 2, 1))
